# baseline (speedup 1.0000x reference)
import functools

import jax
import jax.numpy as jnp
from jax import lax
from jax.experimental import pallas as pl
from jax.experimental.pallas import tpu as pltpu
from jax.experimental.pallas import tpu_sc as plsc

_N_USERS = 1000000
_BATCH = 16384
_NF = 32
_L = 16

_info = plsc.get_sparse_core_info()
_NC, _NS = _info.num_cores, _info.num_subcores
_NW = _NC * _NS
_P = _BATCH // _NW
_CH = 128
_NCH = _P // _CH
_NG = _P // _L

_mesh = plsc.VectorSubcoreMesh(core_axis_name="c", subcore_axis_name="s")


@functools.partial(
    pl.kernel,
    mesh=_mesh,
    out_type=jax.ShapeDtypeStruct((_BATCH,), jnp.float32),
    compiler_params=pltpu.CompilerParams(
        needs_layout_passes=False, use_tc_tiling_on_sc=False),
    scratch_types=[
        pltpu.VMEM((_NCH, _CH), jnp.int32),
        pltpu.VMEM((_NCH, _CH), jnp.int32),
        pltpu.VMEM((_P, _NF), jnp.float32),
        pltpu.VMEM((_P, _NF), jnp.float32),
        pltpu.VMEM((_P,), jnp.float32),
        pltpu.VMEM((_P,), jnp.float32),
        pltpu.VMEM((_P,), jnp.float32),
        pltpu.VMEM((_L,), jnp.float32),
        pltpu.SemaphoreType.DMA,
    ],
)
def _fm_sc(users_hbm, items_hbm, emb_hbm, fc_hbm, bias_hbm, out_hbm,
           idxu_v, idxi_v, eu_v, ei_v, fu_v, fi_v, out_v, bias_v, sem):
    wid = lax.axis_index("s") * _NC + lax.axis_index("c")
    base = wid * _P

    for c in range(_NCH):
        pltpu.sync_copy(users_hbm.at[pl.ds(base + c * _CH, _CH)], idxu_v.at[c])
        pltpu.sync_copy(items_hbm.at[pl.ds(base + c * _CH, _CH)], idxi_v.at[c])
    pltpu.sync_copy(bias_hbm, bias_v.at[pl.ds(0, 1)])

    for c in range(_NCH):
        for j in range(_CH // _L):
            sl = pl.ds(j * _L, _L)
            idxi_v[c, sl] = idxi_v[c, sl] + _N_USERS

    copies = []
    for c in range(_NCH):
        dst = pl.ds(c * _CH, _CH)
        copies.append(pltpu.async_copy(emb_hbm.at[idxu_v.at[c]], eu_v.at[dst], sem))
        copies.append(pltpu.async_copy(emb_hbm.at[idxi_v.at[c]], ei_v.at[dst], sem))
        copies.append(pltpu.async_copy(fc_hbm.at[idxu_v.at[c]], fu_v.at[dst], sem))
        copies.append(pltpu.async_copy(fc_hbm.at[idxi_v.at[c]], fi_v.at[dst], sem))
    for cp in copies:
        cp.wait()

    bias_s = bias_v[...][0]
    lane = lax.iota(jnp.int32, _L)

    def body(g, carry):
        gsl = pl.ds(g * _L, _L)
        acc = fu_v[gsl] + fi_v[gsl] + bias_s
        for j in range(_L):
            s = g * _L + j
            a0 = eu_v[s, pl.ds(0, _L)]
            a1 = eu_v[s, pl.ds(_L, _L)]
            b0 = ei_v[s, pl.ds(0, _L)]
            b1 = ei_v[s, pl.ds(_L, _L)]
            d = jnp.sum(a0 * b0 + a1 * b1)
            acc = jnp.where(lane == j, acc + d, acc)
        out_v[gsl] = acc
        return carry

    lax.fori_loop(0, _NG, body, 0)
    pltpu.sync_copy(out_v, out_hbm.at[pl.ds(base, _P)])


def kernel(users_feat, items_feat, emb_w, fc_w, bias):
    return _fm_sc(users_feat, items_feat, emb_w, fc_w.reshape(-1), bias)
